# baseline (speedup 1.0000x reference)
import jax
import jax.numpy as jnp
from jax import lax
from jax.experimental import pallas as pl
from jax.experimental.pallas import tpu as pltpu
from jax.experimental.pallas import tpu_sc as plsc

NC = 2
NS = 16
NW = NC * NS
L = 16
BW = 128

HER_PROPORTION = 0.8
THRESHOLD = 0.05
TH_SQ = THRESHOLD * THRESHOLD


def _her_body(ach_hbm, des_hbm, rew_hbm, buf_hbm, noise_hbm, idx_hbm,
              goal_out, rew_out,
              idx_v, idx2_v, fut_v, ach_v, des_v, noise_v, rew_v, rewo_v,
              gsem, dsem, osem):
    T = ach_hbm.shape[0]
    D = ach_hbm.shape[1] * ach_hbm.shape[3]
    NCH = BW // L

    wid = lax.axis_index("s") * NC + lax.axis_index("c")
    base = wid * BW

    pltpu.sync_copy(idx_hbm.at[pl.ds(base, BW)], idx_v)

    lane = lax.iota(jnp.int32, L)
    for i in range(NCH):
        iv = idx_v[pl.ds(i * L, L)]
        plsc.store_scatter(idx2_v, [lane + i * L],
                           lax.shift_right_logical(iv, 1))
    gather = pltpu.async_copy(buf_hbm.at[idx2_v], fut_v, gsem)

    dense = []
    for t in range(T):
        for r in range(D // 8):
            dense.append(pltpu.async_copy(
                ach_hbm.at[t, r, wid], ach_v.at[t, pl.ds(r * 8, 8)], dsem))
            dense.append(pltpu.async_copy(
                des_hbm.at[t, r, wid], des_v.at[t, pl.ds(r * 8, 8)], dsem))
    dense.append(pltpu.async_copy(noise_hbm.at[pl.ds(base, BW)], noise_v, dsem))
    dense.append(pltpu.async_copy(rew_hbm.at[pl.ds(wid * T, T)], rew_v, dsem))
    for c in dense:
        c.wait()
    gather.wait()

    for i in range(NCH):
        cond = noise_v[pl.ds(i * L, L)] < HER_PROPORTION
        rows = lane + i * L
        par = (idx_v[pl.ds(i * L, L)] & 1) * D
        accs = [jnp.zeros((L,), jnp.float32) for _ in range(T)]

        def dstep(d, accs, cond=cond, rows=rows, par=par, i=i):
            fut = plsc.load_gather(fut_v, [rows, par + d])
            out = []
            for t in range(T):
                a = ach_v[t, d, pl.ds(i * L, L)]
                de = des_v[t, d, pl.ds(i * L, L)]
                g = jnp.where(cond, fut, de)
                des_v[t, d, pl.ds(i * L, L)] = g
                diff = a - g
                out.append(accs[t] + diff * diff)
            return out

        accs = lax.fori_loop(0, D, dstep, accs, unroll=4)
        for t in range(T):
            nr = -(accs[t] >= TH_SQ).astype(jnp.float32)
            rw = rew_v[t, pl.ds(i * L, L)]
            rewo_v[t, pl.ds(i * L, L)] = jnp.where(cond, nr, rw)

    outs = []
    for t in range(T):
        for r in range(D // 8):
            outs.append(pltpu.async_copy(
                des_v.at[t, pl.ds(r * 8, 8)], goal_out.at[t, r, wid], osem))
    outs.append(pltpu.async_copy(rewo_v, rew_out.at[pl.ds(wid * T, T)], osem))
    for c in outs:
        c.wait()


def _transpose_body(bufT_hbm, out_hbm, slab_v, out_v, tail_v, tout_v,
                    isem, osem):
    D = bufT_hbm.shape[0]
    BUF = bufT_hbm.shape[1]
    NFULL = BUF // 128
    TAIL = BUF - NFULL * 128
    NJ = (NFULL + NW) // NW

    wid = lax.axis_index("s") * NC + lax.axis_index("c")
    lane = lax.iota(jnp.int32, L)

    def block(j, carry):
        c = j * NW + wid

        @pl.when(c < NFULL)
        def _full():
            pltpu.async_copy(bufT_hbm.at[:, pl.ds(c * 128, 128)],
                             slab_v, isem).wait()
            for q in range(64):
                for half in range(2):
                    col = jnp.full((L,), 2 * q + half, jnp.int32)
                    for dc in range(D // L):
                        v = plsc.load_gather(slab_v, [lane + dc * L, col])
                        out_v[q, pl.ds(half * D + dc * L, L)] = v
            pltpu.async_copy(out_v, out_hbm.at[pl.ds(c * 64, 64)], osem).wait()

        @pl.when(c == NFULL)
        def _tail():
            pltpu.async_copy(bufT_hbm.at[:, pl.ds(NFULL * 128, TAIL)],
                             tail_v, isem).wait()
            for q in range(TAIL // 2):
                for half in range(2):
                    col = jnp.full((L,), 2 * q + half, jnp.int32)
                    for dc in range(D // L):
                        v = plsc.load_gather(tail_v, [lane + dc * L, col])
                        tout_v[q, pl.ds(half * D + dc * L, L)] = v
            pltpu.async_copy(tout_v,
                             out_hbm.at[pl.ds(NFULL * 64, TAIL // 2)],
                             osem).wait()

        return carry

    lax.fori_loop(0, NJ, block, 0)


def kernel(achieved_goal, desired_goal, reward, buffer_ag, her_noise, future_idx):
    B, T, D = achieved_goal.shape
    BUF = buffer_ag.shape[0]
    idx32 = future_idx.astype(jnp.int32)

    def to5(x):
        return (x.transpose(1, 2, 0)
                 .reshape(T, D // 8, 8, B // 128, 128)
                 .transpose(0, 1, 3, 2, 4))

    ach5 = to5(achieved_goal)
    des5 = to5(desired_goal)
    rew2 = (reward.reshape(B // 128, 128, T)
                  .transpose(0, 2, 1)
                  .reshape(B // 128 * T, 128))

    mesh = plsc.VectorSubcoreMesh(core_axis_name="c", subcore_axis_name="s",
                                  num_cores=NC, num_subcores=NS)

    transpose_run = pl.kernel(
        _transpose_body,
        out_type=jax.ShapeDtypeStruct((BUF // 2, 2 * D), jnp.float32),
        mesh=mesh,
        compiler_params=pltpu.CompilerParams(needs_layout_passes=False,
                                             use_tc_tiling_on_sc=True),
        scratch_types=[
            pltpu.VMEM((D, 128), jnp.float32),
            pltpu.VMEM((64, 2 * D), jnp.float32),
            pltpu.VMEM((D, 32), jnp.float32),
            pltpu.VMEM((16, 2 * D), jnp.float32),
            pltpu.SemaphoreType.DMA,
            pltpu.SemaphoreType.DMA,
        ],
    )
    buf2 = transpose_run(buffer_ag.T)
    run = pl.kernel(
        _her_body,
        out_type=(
            jax.ShapeDtypeStruct((T, D // 8, B // 128, 8, 128), jnp.float32),
            jax.ShapeDtypeStruct((B // 128 * T, 128), jnp.float32),
        ),
        mesh=mesh,
        compiler_params=pltpu.CompilerParams(needs_layout_passes=False,
                                             use_tc_tiling_on_sc=True),
        scratch_types=[
            pltpu.VMEM((BW,), jnp.int32),
            pltpu.VMEM((BW,), jnp.int32),
            pltpu.VMEM((BW, 2 * D), jnp.float32),
            pltpu.VMEM((T, D, 128), jnp.float32),
            pltpu.VMEM((T, D, 128), jnp.float32),
            pltpu.VMEM((BW,), jnp.float32),
            pltpu.VMEM((T, 128), jnp.float32),
            pltpu.VMEM((T, 128), jnp.float32),
            pltpu.SemaphoreType.DMA,
            pltpu.SemaphoreType.DMA,
            pltpu.SemaphoreType.DMA,
        ],
    )
    goal5, rew2o = run(ach5, des5, rew2, buf2, her_noise, idx32)

    goal = (goal5.transpose(0, 1, 3, 2, 4)
                 .reshape(T, D, B)
                 .transpose(2, 0, 1))
    rew = (rew2o.reshape(B // 128, T, 128)
                .transpose(0, 2, 1)
                .reshape(B, T))
    return goal, rew


# speedup vs baseline: 1.1258x; 1.1258x over previous
import jax
import jax.numpy as jnp
from jax import lax
from jax.experimental import pallas as pl
from jax.experimental.pallas import tpu as pltpu
from jax.experimental.pallas import tpu_sc as plsc

NC = 2
NS = 16
NW = NC * NS
L = 16
BW = 128

HER_PROPORTION = 0.8
THRESHOLD = 0.05
TH_SQ = THRESHOLD * THRESHOLD


def _her_body(ach_hbm, des_hbm, rew_hbm, buf_hbm, noise_hbm, idx_hbm,
              goal_out, rew_out,
              idx_v, idx2_v, fut_v, ach_v, des_v, noise_v, rew_v, rewo_v,
              gsem, dsem, osem):
    T = ach_hbm.shape[0]
    D = ach_hbm.shape[1] * ach_hbm.shape[3]
    NCH = BW // L

    wid = lax.axis_index("s") * NC + lax.axis_index("c")
    base = wid * BW

    pltpu.sync_copy(idx_hbm.at[pl.ds(base, BW)], idx_v)

    lane = lax.iota(jnp.int32, L)
    for i in range(NCH):
        iv = idx_v[pl.ds(i * L, L)]
        plsc.store_scatter(idx2_v, [lane + i * L],
                           lax.shift_right_logical(iv, 1))
    gather = pltpu.async_copy(buf_hbm.at[idx2_v], fut_v, gsem)

    dense = []
    for t in range(T):
        for r in range(D // 8):
            dense.append(pltpu.async_copy(
                ach_hbm.at[t, r, wid], ach_v.at[t, pl.ds(r * 8, 8)], dsem))
            dense.append(pltpu.async_copy(
                des_hbm.at[t, r, wid], des_v.at[t, pl.ds(r * 8, 8)], dsem))
    dense.append(pltpu.async_copy(noise_hbm.at[pl.ds(base, BW)], noise_v, dsem))
    dense.append(pltpu.async_copy(rew_hbm.at[pl.ds(wid * T, T)], rew_v, dsem))
    for c in dense:
        c.wait()
    gather.wait()

    for i in range(NCH):
        cond = noise_v[pl.ds(i * L, L)] < HER_PROPORTION
        rows = lane + i * L
        par = (idx_v[pl.ds(i * L, L)] & 1) * D
        accs = [jnp.zeros((L,), jnp.float32) for _ in range(T)]

        def dstep(d, accs, cond=cond, rows=rows, par=par, i=i):
            fut = plsc.load_gather(fut_v, [rows, par + d])
            out = []
            for t in range(T):
                a = ach_v[t, d, pl.ds(i * L, L)]
                de = des_v[t, d, pl.ds(i * L, L)]
                g = jnp.where(cond, fut, de)
                des_v[t, d, pl.ds(i * L, L)] = g
                diff = a - g
                out.append(accs[t] + diff * diff)
            return out

        accs = lax.fori_loop(0, D, dstep, accs, unroll=4)
        for t in range(T):
            nr = -(accs[t] >= TH_SQ).astype(jnp.float32)
            rw = rew_v[t, pl.ds(i * L, L)]
            rewo_v[t, pl.ds(i * L, L)] = jnp.where(cond, nr, rw)

    outs = []
    for t in range(T):
        for r in range(D // 8):
            outs.append(pltpu.async_copy(
                des_v.at[t, pl.ds(r * 8, 8)], goal_out.at[t, r, wid], osem))
    outs.append(pltpu.async_copy(rewo_v, rew_out.at[pl.ds(wid * T, T)], osem))
    for c in outs:
        c.wait()


def _transpose_body(bufT_hbm, out_hbm, slab_v, out_v, tail_v, tout_v,
                    isem, osem):
    D = bufT_hbm.shape[0]
    BUF = bufT_hbm.shape[1]
    NFULL = BUF // 128
    TAIL = BUF - NFULL * 128
    NJ = (NFULL + NW - 1) // NW

    wid = lax.axis_index("s") * NC + lax.axis_index("c")
    lane = lax.iota(jnp.int32, L)
    rows = [lane + dc * L for dc in range(D // L)]
    pvec = [jnp.full((L,), p, jnp.int32) for p in range(2)]

    def start_in(j, p):
        c = j * NW + wid

        @pl.when(c < NFULL)
        def _():
            pltpu.async_copy(bufT_hbm.at[:, pl.ds(c * 128, 128)],
                             slab_v.at[p], isem)

    def xpose(j, p):
        c = j * NW + wid

        @pl.when(c < NFULL)
        def _():
            pltpu.make_async_copy(bufT_hbm.at[:, pl.ds(0, 128)],
                                  slab_v.at[p], isem).wait()
            for q in range(64):
                for half in range(2):
                    col = jnp.full((L,), 2 * q + half, jnp.int32)
                    for dc in range(D // L):
                        v = plsc.load_gather(slab_v, [pvec[p], rows[dc], col])
                        out_v[p, q, pl.ds(half * D + dc * L, L)] = v
            pltpu.async_copy(out_v.at[p], out_hbm.at[pl.ds(c * 64, 64)], osem)

    def wait_out(j, p):
        c = j * NW + wid

        @pl.when(c < NFULL)
        def _():
            pltpu.make_async_copy(out_v.at[p],
                                  out_hbm.at[pl.ds(0, 64)], osem).wait()

    start_in(0, 0)

    def block(jj, carry):
        j = jj * 2
        start_in(j + 1, 1)
        xpose(j, 0)
        start_in(j + 2, 0)
        xpose(j + 1, 1)
        wait_out(j, 0)
        wait_out(j + 1, 1)
        return carry

    lax.fori_loop(0, (NJ + 1) // 2, block, 0)

    @pl.when(wid == NW - 1)
    def _tail():
        pltpu.async_copy(bufT_hbm.at[:, pl.ds(NFULL * 128, TAIL)],
                         tail_v, isem).wait()
        for q in range(TAIL // 2):
            for half in range(2):
                col = jnp.full((L,), 2 * q + half, jnp.int32)
                for dc in range(D // L):
                    v = plsc.load_gather(tail_v, [rows[dc], col])
                    tout_v[q, pl.ds(half * D + dc * L, L)] = v
        pltpu.async_copy(tout_v,
                         out_hbm.at[pl.ds(NFULL * 64, TAIL // 2)],
                         osem).wait()


def kernel(achieved_goal, desired_goal, reward, buffer_ag, her_noise, future_idx):
    B, T, D = achieved_goal.shape
    BUF = buffer_ag.shape[0]
    idx32 = future_idx.astype(jnp.int32)

    def to5(x):
        return (x.transpose(1, 2, 0)
                 .reshape(T, D // 8, 8, B // 128, 128)
                 .transpose(0, 1, 3, 2, 4))

    ach5 = to5(achieved_goal)
    des5 = to5(desired_goal)
    rew2 = (reward.reshape(B // 128, 128, T)
                  .transpose(0, 2, 1)
                  .reshape(B // 128 * T, 128))

    mesh = plsc.VectorSubcoreMesh(core_axis_name="c", subcore_axis_name="s",
                                  num_cores=NC, num_subcores=NS)

    transpose_run = pl.kernel(
        _transpose_body,
        out_type=jax.ShapeDtypeStruct((BUF // 2, 2 * D), jnp.float32),
        mesh=mesh,
        compiler_params=pltpu.CompilerParams(needs_layout_passes=False,
                                             use_tc_tiling_on_sc=True),
        scratch_types=[
            pltpu.VMEM((2, D, 128), jnp.float32),
            pltpu.VMEM((2, 64, 2 * D), jnp.float32),
            pltpu.VMEM((D, 32), jnp.float32),
            pltpu.VMEM((16, 2 * D), jnp.float32),
            pltpu.SemaphoreType.DMA,
            pltpu.SemaphoreType.DMA,
        ],
    )
    buf2 = transpose_run(buffer_ag.T)
    run = pl.kernel(
        _her_body,
        out_type=(
            jax.ShapeDtypeStruct((T, D // 8, B // 128, 8, 128), jnp.float32),
            jax.ShapeDtypeStruct((B // 128 * T, 128), jnp.float32),
        ),
        mesh=mesh,
        compiler_params=pltpu.CompilerParams(needs_layout_passes=False,
                                             use_tc_tiling_on_sc=True),
        scratch_types=[
            pltpu.VMEM((BW,), jnp.int32),
            pltpu.VMEM((BW,), jnp.int32),
            pltpu.VMEM((BW, 2 * D), jnp.float32),
            pltpu.VMEM((T, D, 128), jnp.float32),
            pltpu.VMEM((T, D, 128), jnp.float32),
            pltpu.VMEM((BW,), jnp.float32),
            pltpu.VMEM((T, 128), jnp.float32),
            pltpu.VMEM((T, 128), jnp.float32),
            pltpu.SemaphoreType.DMA,
            pltpu.SemaphoreType.DMA,
            pltpu.SemaphoreType.DMA,
        ],
    )
    goal5, rew2o = run(ach5, des5, rew2, buf2, her_noise, idx32)

    goal = (goal5.transpose(0, 1, 3, 2, 4)
                 .reshape(T, D, B)
                 .transpose(2, 0, 1))
    rew = (rew2o.reshape(B // 128, T, 128)
                .transpose(0, 2, 1)
                .reshape(B, T))
    return goal, rew


# speedup vs baseline: 1.1683x; 1.0377x over previous
import jax
import jax.numpy as jnp
from jax import lax
from jax.experimental import pallas as pl
from jax.experimental.pallas import tpu as pltpu
from jax.experimental.pallas import tpu_sc as plsc

NC = 2
NS = 16
NW = NC * NS
L = 16
BW = 128

HER_PROPORTION = 0.8
THRESHOLD = 0.05
TH_SQ = THRESHOLD * THRESHOLD


def _her_body(ach_hbm, des_hbm, rew_hbm, buf_hbm, noise_hbm, idx_hbm,
              goal_out, rew_out,
              idx_v, idx2_v, fut_v, ach_v, des_v, noise_v, rew_v, rewo_v,
              gsem, dsem, osem):
    T = ach_hbm.shape[0]
    D = ach_hbm.shape[1] * ach_hbm.shape[3]
    NCH = BW // L

    wid = lax.axis_index("s") * NC + lax.axis_index("c")
    base = wid * BW

    pltpu.sync_copy(idx_hbm.at[pl.ds(base, BW)], idx_v)

    lane = lax.iota(jnp.int32, L)
    for i in range(NCH):
        iv = idx_v[pl.ds(i * L, L)]
        plsc.store_scatter(idx2_v, [lane + i * L],
                           lax.shift_right_logical(iv, 1))
    gather = pltpu.async_copy(buf_hbm.at[idx2_v],
                              fut_v.at[:, pl.ds(0, 2 * D)], gsem)

    dense = []
    for t in range(T):
        for r in range(D // 8):
            dense.append(pltpu.async_copy(
                ach_hbm.at[t, r, wid], ach_v.at[t, pl.ds(r * 8, 8)], dsem))
            dense.append(pltpu.async_copy(
                des_hbm.at[t, r, wid], des_v.at[t, pl.ds(r * 8, 8)], dsem))
    dense.append(pltpu.async_copy(noise_hbm.at[pl.ds(base, BW)], noise_v, dsem))
    dense.append(pltpu.async_copy(rew_hbm.at[pl.ds(wid * T, T)], rew_v, dsem))
    for c in dense:
        c.wait()
    gather.wait()

    for i in range(NCH):
        cond = noise_v[pl.ds(i * L, L)] < HER_PROPORTION
        rows = lane + i * L
        par = (idx_v[pl.ds(i * L, L)] & 1) * D
        accs = [jnp.zeros((L,), jnp.float32) for _ in range(T)]

        def dstep(d, accs, cond=cond, rows=rows, par=par, i=i):
            fut = plsc.load_gather(fut_v, [rows, par + d])
            out = []
            for t in range(T):
                a = ach_v[t, d, pl.ds(i * L, L)]
                de = des_v[t, d, pl.ds(i * L, L)]
                g = jnp.where(cond, fut, de)
                des_v[t, d, pl.ds(i * L, L)] = g
                diff = a - g
                out.append(accs[t] + diff * diff)
            return out

        accs = lax.fori_loop(0, D, dstep, accs, unroll=4)
        for t in range(T):
            nr = -(accs[t] >= TH_SQ).astype(jnp.float32)
            rw = rew_v[t, pl.ds(i * L, L)]
            rewo_v[t, pl.ds(i * L, L)] = jnp.where(cond, nr, rw)

    outs = []
    for t in range(T):
        for r in range(D // 8):
            outs.append(pltpu.async_copy(
                des_v.at[t, pl.ds(r * 8, 8)], goal_out.at[t, r, wid], osem))
    outs.append(pltpu.async_copy(rewo_v, rew_out.at[pl.ds(wid * T, T)], osem))
    for c in outs:
        c.wait()


def _transpose_body(bufT_hbm, btail_hbm, out_hbm, slab_v, out_v, tout_v,
                    isem, osem):
    D = bufT_hbm.shape[0]
    BUF = bufT_hbm.shape[1]
    NFULL = BUF // 128
    TAIL = BUF - NFULL * 128
    NJ = (NFULL + NW - 1) // NW

    wid = lax.axis_index("s") * NC + lax.axis_index("c")
    lane = lax.iota(jnp.int32, L)
    rows = [lane + dc * L for dc in range(D // L)]
    pvec = [jnp.full((L,), p, jnp.int32) for p in range(2)]

    def start_in(j, p):
        c = j * NW + wid

        @pl.when(c < NFULL)
        def _():
            pltpu.async_copy(bufT_hbm.at[:, pl.ds(c * 128, 128)],
                             slab_v.at[p, :, pl.ds(0, 128)], isem)

    def xpose(j, p):
        c = j * NW + wid

        @pl.when(c < NFULL)
        def _():
            pltpu.make_async_copy(bufT_hbm.at[:, pl.ds(0, 128)],
                                  slab_v.at[p, :, pl.ds(0, 128)], isem).wait()

            def qstep(q, carry):
                for half in range(2):
                    col = jnp.broadcast_to(2 * q + half, (L,))
                    for dc in range(D // L):
                        v = plsc.load_gather(slab_v, [pvec[p], rows[dc], col])
                        out_v[p, q, pl.ds(half * D + dc * L, L)] = v
                return carry

            lax.fori_loop(0, 64, qstep, 0, unroll=4)
            pltpu.async_copy(out_v.at[p], out_hbm.at[pl.ds(c * 64, 64)], osem)

    def wait_out(j, p):
        c = j * NW + wid

        @pl.when(c < NFULL)
        def _():
            pltpu.make_async_copy(out_v.at[p],
                                  out_hbm.at[pl.ds(0, 64)], osem).wait()

    start_in(0, 0)

    def block(jj, carry):
        j = jj * 2
        start_in(j + 1, 1)
        xpose(j, 0)
        start_in(j + 2, 0)
        xpose(j + 1, 1)
        wait_out(j, 0)
        wait_out(j + 1, 1)
        return carry

    lax.fori_loop(0, (NJ + 1) // 2, block, 0)

    @pl.when(wid == NW - 1)
    def _tail():
        pltpu.async_copy(btail_hbm, tout_v, isem).wait()
        pltpu.async_copy(tout_v,
                         out_hbm.at[pl.ds(NFULL * 64, TAIL // 2)],
                         osem).wait()


def kernel(achieved_goal, desired_goal, reward, buffer_ag, her_noise, future_idx):
    B, T, D = achieved_goal.shape
    BUF = buffer_ag.shape[0]
    idx32 = future_idx.astype(jnp.int32)

    def to5(x):
        return (x.transpose(1, 2, 0)
                 .reshape(T, D // 8, 8, B // 128, 128)
                 .transpose(0, 1, 3, 2, 4))

    ach5 = to5(achieved_goal)
    des5 = to5(desired_goal)
    rew2 = (reward.reshape(B // 128, 128, T)
                  .transpose(0, 2, 1)
                  .reshape(B // 128 * T, 128))

    mesh = plsc.VectorSubcoreMesh(core_axis_name="c", subcore_axis_name="s",
                                  num_cores=NC, num_subcores=NS)

    transpose_run = pl.kernel(
        _transpose_body,
        out_type=jax.ShapeDtypeStruct((BUF // 2, 2 * D), jnp.float32),
        mesh=mesh,
        compiler_params=pltpu.CompilerParams(needs_layout_passes=False,
                                             use_tc_tiling_on_sc=True),
        scratch_types=[
            pltpu.VMEM((2, D, 129), jnp.float32),
            pltpu.VMEM((2, 64, 2 * D), jnp.float32),
            pltpu.VMEM((16, 2 * D), jnp.float32),
            pltpu.SemaphoreType.DMA,
            pltpu.SemaphoreType.DMA,
        ],
    )
    btail = buffer_ag[BUF - 32:].reshape(16, 2 * D)
    buf2 = transpose_run(buffer_ag.T, btail)
    run = pl.kernel(
        _her_body,
        out_type=(
            jax.ShapeDtypeStruct((T, D // 8, B // 128, 8, 128), jnp.float32),
            jax.ShapeDtypeStruct((B // 128 * T, 128), jnp.float32),
        ),
        mesh=mesh,
        compiler_params=pltpu.CompilerParams(needs_layout_passes=False,
                                             use_tc_tiling_on_sc=True),
        scratch_types=[
            pltpu.VMEM((BW,), jnp.int32),
            pltpu.VMEM((BW,), jnp.int32),
            pltpu.VMEM((BW, 2 * D + 1), jnp.float32),
            pltpu.VMEM((T, D, 128), jnp.float32),
            pltpu.VMEM((T, D, 128), jnp.float32),
            pltpu.VMEM((BW,), jnp.float32),
            pltpu.VMEM((T, 128), jnp.float32),
            pltpu.VMEM((T, 128), jnp.float32),
            pltpu.SemaphoreType.DMA,
            pltpu.SemaphoreType.DMA,
            pltpu.SemaphoreType.DMA,
        ],
    )
    goal5, rew2o = run(ach5, des5, rew2, buf2, her_noise, idx32)

    goal = (goal5.transpose(0, 1, 3, 2, 4)
                 .reshape(T, D, B)
                 .transpose(2, 0, 1))
    rew = (rew2o.reshape(B // 128, T, 128)
                .transpose(0, 2, 1)
                .reshape(B, T))
    return goal, rew


# speedup vs baseline: 2.7002x; 2.3113x over previous
import jax
import jax.numpy as jnp
from jax import lax
from jax.experimental import pallas as pl
from jax.experimental.pallas import tpu as pltpu
from jax.experimental.pallas import tpu_sc as plsc

NC = 2
NS = 16
NW = NC * NS
L = 16
BW = 128

HER_PROPORTION = 0.8
THRESHOLD = 0.05
TH_SQ = THRESHOLD * THRESHOLD


def _her_body(ach_hbm, des_hbm, rew_hbm, buf_hbm, noise_hbm, idx_hbm,
              goal_out, rew_out,
              idx_v, fut_v, ach_v, des_v, noise_v, rew_v, rewo_v,
              gsem, dsem, osem):
    T = ach_hbm.shape[0]
    D = ach_hbm.shape[1] * ach_hbm.shape[3]
    NCH = BW // L

    wid = lax.axis_index("s") * NC + lax.axis_index("c")
    base = wid * BW

    pltpu.sync_copy(idx_hbm.at[pl.ds(base, BW)], idx_v)

    lane = lax.iota(jnp.int32, L)
    gather = pltpu.async_copy(buf_hbm.at[idx_v],
                              fut_v.at[:, pl.ds(0, 2 * D)], gsem)

    dense = []
    for t in range(T):
        for r in range(D // 8):
            dense.append(pltpu.async_copy(
                ach_hbm.at[t, r, wid], ach_v.at[t, pl.ds(r * 8, 8)], dsem))
            dense.append(pltpu.async_copy(
                des_hbm.at[t, r, wid], des_v.at[t, pl.ds(r * 8, 8)], dsem))
    dense.append(pltpu.async_copy(noise_hbm.at[pl.ds(base, BW)], noise_v, dsem))
    dense.append(pltpu.async_copy(rew_hbm.at[pl.ds(wid * T, T)], rew_v, dsem))
    for c in dense:
        c.wait()
    gather.wait()

    for i in range(NCH):
        cond = noise_v[pl.ds(i * L, L)] < HER_PROPORTION
        rows = lane + i * L
        accs = [jnp.zeros((L,), jnp.float32) for _ in range(T)]

        def dstep(d, accs, cond=cond, rows=rows, i=i):
            fut = plsc.load_gather(fut_v, [rows, jnp.broadcast_to(d, (L,))])
            out = []
            for t in range(T):
                a = ach_v[t, d, pl.ds(i * L, L)]
                de = des_v[t, d, pl.ds(i * L, L)]
                g = jnp.where(cond, fut, de)
                des_v[t, d, pl.ds(i * L, L)] = g
                diff = a - g
                out.append(accs[t] + diff * diff)
            return out

        accs = lax.fori_loop(0, D, dstep, accs, unroll=4)
        for t in range(T):
            nr = -(accs[t] >= TH_SQ).astype(jnp.float32)
            rw = rew_v[t, pl.ds(i * L, L)]
            rewo_v[t, pl.ds(i * L, L)] = jnp.where(cond, nr, rw)

    outs = []
    for t in range(T):
        for r in range(D // 8):
            outs.append(pltpu.async_copy(
                des_v.at[t, pl.ds(r * 8, 8)], goal_out.at[t, r, wid], osem))
    outs.append(pltpu.async_copy(rewo_v, rew_out.at[pl.ds(wid * T, T)], osem))
    for c in outs:
        c.wait()


def kernel(achieved_goal, desired_goal, reward, buffer_ag, her_noise, future_idx):
    B, T, D = achieved_goal.shape
    BUF = buffer_ag.shape[0]
    idx32 = future_idx.astype(jnp.int32)

    def to5(x):
        return (x.transpose(1, 2, 0)
                 .reshape(T, D // 8, 8, B // 128, 128)
                 .transpose(0, 1, 3, 2, 4))

    ach5 = to5(achieved_goal)
    des5 = to5(desired_goal)
    rew2 = (reward.reshape(B // 128, 128, T)
                  .transpose(0, 2, 1)
                  .reshape(B // 128 * T, 128))

    mesh = plsc.VectorSubcoreMesh(core_axis_name="c", subcore_axis_name="s",
                                  num_cores=NC, num_subcores=NS)

    buf2 = jnp.pad(buffer_ag, ((0, 0), (0, D)))
    run = pl.kernel(
        _her_body,
        out_type=(
            jax.ShapeDtypeStruct((T, D // 8, B // 128, 8, 128), jnp.float32),
            jax.ShapeDtypeStruct((B // 128 * T, 128), jnp.float32),
        ),
        mesh=mesh,
        compiler_params=pltpu.CompilerParams(needs_layout_passes=False,
                                             use_tc_tiling_on_sc=True),
        scratch_types=[
            pltpu.VMEM((BW,), jnp.int32),
            pltpu.VMEM((BW, 2 * D + 1), jnp.float32),
            pltpu.VMEM((T, D, 128), jnp.float32),
            pltpu.VMEM((T, D, 128), jnp.float32),
            pltpu.VMEM((BW,), jnp.float32),
            pltpu.VMEM((T, 128), jnp.float32),
            pltpu.VMEM((T, 128), jnp.float32),
            pltpu.SemaphoreType.DMA,
            pltpu.SemaphoreType.DMA,
            pltpu.SemaphoreType.DMA,
        ],
    )
    goal5, rew2o = run(ach5, des5, rew2, buf2, her_noise, idx32)

    goal = (goal5.transpose(0, 1, 3, 2, 4)
                 .reshape(T, D, B)
                 .transpose(2, 0, 1))
    rew = (rew2o.reshape(B // 128, T, 128)
                .transpose(0, 2, 1)
                .reshape(B, T))
    return goal, rew
